# baseline (speedup 1.0000x reference)
import jax
import jax.numpy as jnp
from jax import lax
from jax.experimental import pallas as pl
from jax.experimental.pallas import tpu as pltpu
from jax.experimental.pallas import tpu_sc as plsc

BATCH = 4096
HIST = 200
FEATURE_DIM = 100000
EMBED = 128
NUM_WORKERS = 32
BPW = BATCH // NUM_WORKERS
ZROW = FEATURE_DIM + 1
XSTRIDE = 208

_OFFS = tuple(range(0, 192, 16)) + (184,)


def _sc_body(x_hbm, xt_hbm, neg_hbm, tab_hbm, q_out, ub_out, bl_out, nub_out,
             xbuf, xtbuf, nbuf, ebuf0, ebuf1, qidx, cnts, sem, sem0, sem1,
             semw0, semw1):
    wid = lax.axis_index("s") * 2 + lax.axis_index("c")
    base = wid * BPW

    pltpu.sync_copy(x_hbm.at[pl.ds(base * XSTRIDE, BPW * XSTRIDE)], xbuf)
    pltpu.sync_copy(xt_hbm.at[pl.ds(base * XSTRIDE, BPW * XSTRIDE)], xtbuf)
    pltpu.sync_copy(neg_hbm.at[pl.ds(base * HIST, BPW * HIST)], nbuf)

    zeros16 = jnp.zeros((16,), jnp.int32)
    ones16 = jnp.full((16,), 1, jnp.int32)
    zrow16 = jnp.full((16,), ZROW, jnp.int32)

    for c in range(BPW // 16):
        cnts[pl.ds(c * 16, 16)] = zeros16

    def cstep(j, carry):
        for c in range(BPW // 16):
            v = xtbuf[pl.ds(j * BPW + c * 16, 16)]
            cnts[pl.ds(c * 16, 16)] = (
                cnts[pl.ds(c * 16, 16)] + jnp.where(v > zeros16, ones16,
                                                    zeros16))
        return carry

    lax.fori_loop(0, HIST, cstep, 0)
    pltpu.sync_copy(cnts, bl_out.at[pl.ds(base, BPW)])

    for c in range(BPW // 16):
        qidx[pl.ds(c * 16, 16)] = xtbuf[pl.ds(HIST * BPW + c * 16, 16)]

    def remap_x(b, carry):
        for off in _OFFS:
            v = xbuf[pl.ds(b * XSTRIDE + off, 16)]
            xbuf[pl.ds(b * XSTRIDE + off, 16)] = jnp.where(v > zeros16, v,
                                                           zrow16)
        return carry

    lax.fori_loop(0, BPW, remap_x, 0)

    def remap_n(b, carry):
        for off in _OFFS:
            v = nbuf[pl.ds(b * HIST + off, 16)]
            nbuf[pl.ds(b * HIST + off, 16)] = jnp.where(v > zeros16, v, zrow16)
        return carry

    lax.fori_loop(0, BPW, remap_n, 0)

    pltpu.async_copy(tab_hbm.at[qidx], ebuf0.at[pl.ds(0, BPW), :], sem).wait()
    pltpu.sync_copy(ebuf0.at[pl.ds(0, BPW), :], q_out.at[pl.ds(base, BPW)])

    def start_g(idx_ref, r, buf, sem_):
        pltpu.async_copy(tab_hbm.at[idx_ref.at[pl.ds(r, 104)]],
                         buf.at[pl.ds(0, 104), :], sem_)
        pltpu.async_copy(tab_hbm.at[idx_ref.at[pl.ds(r + 104, 96)]],
                         buf.at[pl.ds(104, 96), :], sem_)

    def drain_g(buf, sem_):
        pltpu.make_async_copy(tab_hbm.at[pl.ds(0, HIST), :], buf, sem_).wait()

    def emit(idx_ref, stride, out_ref):
        start_g(idx_ref, 0, ebuf0, sem0)
        start_g(idx_ref, stride, ebuf1, sem1)

        def pair(i, carry):
            b = 2 * i
            drain_g(ebuf0, sem0)
            pltpu.async_copy(ebuf0, out_ref.at[base + b], semw0)
            drain_g(ebuf1, sem1)
            pltpu.async_copy(ebuf1, out_ref.at[base + b + 1], semw1)

            pltpu.make_async_copy(ebuf0, out_ref.at[base + b], semw0).wait()

            @pl.when(b + 2 < BPW)
            def _():
                start_g(idx_ref, (b + 2) * stride, ebuf0, sem0)

            pltpu.make_async_copy(ebuf1, out_ref.at[base + b + 1],
                                  semw1).wait()

            @pl.when(b + 3 < BPW)
            def _():
                start_g(idx_ref, (b + 3) * stride, ebuf1, sem1)

            return carry

        lax.fori_loop(0, BPW // 2, pair, 0)

    emit(xbuf, XSTRIDE, ub_out)
    emit(nbuf, HIST, nub_out)


@jax.jit
def _impl(x, neg_x, table):
    tab2 = jnp.concatenate(
        [table, jnp.zeros((1, EMBED), jnp.float32)], axis=0)
    xpad = jnp.pad(x, ((0, 0), (0, XSTRIDE - (HIST + 1))))
    xflat = xpad.reshape(-1)
    xtr = (xpad.T.reshape(XSTRIDE, NUM_WORKERS, BPW)
           .transpose(1, 0, 2).reshape(-1))
    negf = neg_x.reshape(-1)
    fn = pl.kernel(
        _sc_body,
        out_type=(
            jax.ShapeDtypeStruct((BATCH, EMBED), jnp.float32),
            jax.ShapeDtypeStruct((BATCH, HIST, EMBED), jnp.float32),
            jax.ShapeDtypeStruct((BATCH,), jnp.int32),
            jax.ShapeDtypeStruct((BATCH, HIST, EMBED), jnp.float32),
        ),
        mesh=plsc.VectorSubcoreMesh(core_axis_name="c", subcore_axis_name="s"),
        scratch_types=[
            pltpu.VMEM((BPW * XSTRIDE,), jnp.int32),
            pltpu.VMEM((BPW * XSTRIDE,), jnp.int32),
            pltpu.VMEM((BPW * HIST,), jnp.int32),
            pltpu.VMEM((HIST, EMBED), jnp.float32),
            pltpu.VMEM((HIST, EMBED), jnp.float32),
            pltpu.VMEM((BPW,), jnp.int32),
            pltpu.VMEM((BPW,), jnp.int32),
            pltpu.SemaphoreType.DMA,
            pltpu.SemaphoreType.DMA,
            pltpu.SemaphoreType.DMA,
            pltpu.SemaphoreType.DMA,
            pltpu.SemaphoreType.DMA,
        ],
    )
    q, ub, bl, nub = fn(xflat, xtr, negf, tab2)
    return q.reshape(BATCH, 1, EMBED), ub, bl, nub


def kernel(x, neg_x, table):
    return _impl(x, neg_x, table)


# speedup vs baseline: 1.0180x; 1.0180x over previous
import jax
import jax.numpy as jnp
from jax import lax
from jax.experimental import pallas as pl
from jax.experimental.pallas import tpu as pltpu
from jax.experimental.pallas import tpu_sc as plsc

BATCH = 4096
HIST = 200
FEATURE_DIM = 100000
EMBED = 128
NUM_WORKERS = 32
BPW = BATCH // NUM_WORKERS
ZROW = FEATURE_DIM + 1
XSTRIDE = 208

_OFFS = tuple(range(0, 192, 16)) + (184,)


def _sc_body(x_hbm, xt_hbm, neg_hbm, tab_hbm, q_out, ub_out, bl_out, nub_out,
             xbuf, xtbuf, nbuf, ebuf0, ebuf1, qidx, cnts, sem, sem0, sem1,
             semw0, semw1):
    wid = lax.axis_index("s") * 2 + lax.axis_index("c")
    base = wid * BPW

    pltpu.sync_copy(x_hbm.at[pl.ds(base * XSTRIDE, BPW * XSTRIDE)], xbuf)
    pltpu.sync_copy(xt_hbm.at[pl.ds(base * XSTRIDE, BPW * XSTRIDE)], xtbuf)
    pltpu.sync_copy(neg_hbm.at[pl.ds(base * HIST, BPW * HIST)], nbuf)

    zeros16 = jnp.zeros((16,), jnp.int32)
    ones16 = jnp.full((16,), 1, jnp.int32)
    zrow16 = jnp.full((16,), ZROW, jnp.int32)

    for c in range(BPW // 16):
        cnts[pl.ds(c * 16, 16)] = zeros16

    def cstep(j):
        for c in range(BPW // 16):
            v = xtbuf[pl.ds(j * BPW + c * 16, 16)]
            cnts[pl.ds(c * 16, 16)] = (
                cnts[pl.ds(c * 16, 16)] + jnp.where(v > zeros16, ones16,
                                                    zeros16))

    for c in range(BPW // 16):
        qidx[pl.ds(c * 16, 16)] = xtbuf[pl.ds(HIST * BPW + c * 16, 16)]

    def remap_row(ref, r):
        for off in _OFFS:
            v = ref[pl.ds(r + off, 16)]
            ref[pl.ds(r + off, 16)] = jnp.where(v > zeros16, v, zrow16)

    def remap_x(b, carry):
        remap_row(xbuf, b * XSTRIDE)
        return carry

    lax.fori_loop(0, BPW, remap_x, 0)

    pltpu.async_copy(tab_hbm.at[qidx], ebuf0.at[pl.ds(0, BPW), :], sem).wait()
    pltpu.sync_copy(ebuf0.at[pl.ds(0, BPW), :], q_out.at[pl.ds(base, BPW)])

    def start_g(idx_ref, r, buf, sem_):
        pltpu.async_copy(tab_hbm.at[idx_ref.at[pl.ds(r, 104)]],
                         buf.at[pl.ds(0, 104), :], sem_)
        pltpu.async_copy(tab_hbm.at[idx_ref.at[pl.ds(r + 104, 96)]],
                         buf.at[pl.ds(104, 96), :], sem_)

    def drain_g(buf, sem_):
        pltpu.make_async_copy(tab_hbm.at[pl.ds(0, HIST), :], buf, sem_).wait()

    def emit(idx_ref, stride, out_ref, hidden):
        start_g(idx_ref, 0, ebuf0, sem0)
        start_g(idx_ref, stride, ebuf1, sem1)

        def pair(i, carry):
            b = 2 * i
            drain_g(ebuf0, sem0)
            pltpu.async_copy(ebuf0, out_ref.at[base + b], semw0)
            drain_g(ebuf1, sem1)
            pltpu.async_copy(ebuf1, out_ref.at[base + b + 1], semw1)

            hidden(i)

            pltpu.make_async_copy(ebuf0, out_ref.at[base + b], semw0).wait()

            @pl.when(b + 2 < BPW)
            def _():
                start_g(idx_ref, (b + 2) * stride, ebuf0, sem0)

            pltpu.make_async_copy(ebuf1, out_ref.at[base + b + 1],
                                  semw1).wait()

            @pl.when(b + 3 < BPW)
            def _():
                start_g(idx_ref, (b + 3) * stride, ebuf1, sem1)

            return carry

        lax.fori_loop(0, BPW // 2, pair, 0)

    def user_hidden(i):
        remap_row(nbuf, (2 * i) * HIST)
        remap_row(nbuf, (2 * i + 1) * HIST)
        for t in range(4):
            j = 4 * i + t

            @pl.when(j < HIST)
            def _():
                cstep(j)

    emit(xbuf, XSTRIDE, ub_out, user_hidden)
    pltpu.sync_copy(cnts, bl_out.at[pl.ds(base, BPW)])
    emit(nbuf, HIST, nub_out, lambda i: None)


@jax.jit
def _impl(x, neg_x, table):
    tab2 = jnp.concatenate(
        [table, jnp.zeros((1, EMBED), jnp.float32)], axis=0)
    xpad = jnp.pad(x, ((0, 0), (0, XSTRIDE - (HIST + 1))))
    xflat = xpad.reshape(-1)
    xtr = (xpad.T.reshape(XSTRIDE, NUM_WORKERS, BPW)
           .transpose(1, 0, 2).reshape(-1))
    negf = neg_x.reshape(-1)
    fn = pl.kernel(
        _sc_body,
        out_type=(
            jax.ShapeDtypeStruct((BATCH, EMBED), jnp.float32),
            jax.ShapeDtypeStruct((BATCH, HIST, EMBED), jnp.float32),
            jax.ShapeDtypeStruct((BATCH,), jnp.int32),
            jax.ShapeDtypeStruct((BATCH, HIST, EMBED), jnp.float32),
        ),
        mesh=plsc.VectorSubcoreMesh(core_axis_name="c", subcore_axis_name="s"),
        scratch_types=[
            pltpu.VMEM((BPW * XSTRIDE,), jnp.int32),
            pltpu.VMEM((BPW * XSTRIDE,), jnp.int32),
            pltpu.VMEM((BPW * HIST,), jnp.int32),
            pltpu.VMEM((HIST, EMBED), jnp.float32),
            pltpu.VMEM((HIST, EMBED), jnp.float32),
            pltpu.VMEM((BPW,), jnp.int32),
            pltpu.VMEM((BPW,), jnp.int32),
            pltpu.SemaphoreType.DMA,
            pltpu.SemaphoreType.DMA,
            pltpu.SemaphoreType.DMA,
            pltpu.SemaphoreType.DMA,
            pltpu.SemaphoreType.DMA,
        ],
    )
    q, ub, bl, nub = fn(xflat, xtr, negf, tab2)
    return q.reshape(BATCH, 1, EMBED), ub, bl, nub


def kernel(x, neg_x, table):
    return _impl(x, neg_x, table)


# speedup vs baseline: 1.0200x; 1.0020x over previous
import jax
import jax.numpy as jnp
from jax import lax
from jax.experimental import pallas as pl
from jax.experimental.pallas import tpu as pltpu
from jax.experimental.pallas import tpu_sc as plsc

BATCH = 4096
HIST = 200
FEATURE_DIM = 100000
EMBED = 128
NUM_WORKERS = 32
BPW = BATCH // NUM_WORKERS
ZROW = FEATURE_DIM + 1
XSTRIDE = 208
IPW = BPW * HIST
CH = IPW // BPW
NRING = 3


def _sc_body(xb_hbm, xt_hbm, neg_hbm, tab_hbm, q_out, ub_out, bl_out, nub_out,
             xbuf, xtbuf, nbuf, ebufs, qidx, cnts, sem, gsems, wsems):
    wid = lax.axis_index("s") * 2 + lax.axis_index("c")
    base = wid * BPW

    pltpu.sync_copy(xb_hbm.at[pl.ds(base * HIST, IPW)], xbuf)
    pltpu.sync_copy(xt_hbm.at[pl.ds(base * XSTRIDE, BPW * XSTRIDE)], xtbuf)
    pltpu.sync_copy(neg_hbm.at[pl.ds(base * HIST, IPW)], nbuf)

    zeros16 = jnp.zeros((16,), jnp.int32)
    ones16 = jnp.full((16,), 1, jnp.int32)
    zrow16 = jnp.full((16,), ZROW, jnp.int32)

    for c in range(BPW // 16):
        cnts[pl.ds(c * 16, 16)] = zeros16

    def cstep(j):
        for c in range(BPW // 16):
            v = xtbuf[pl.ds(j * BPW + c * 16, 16)]
            cnts[pl.ds(c * 16, 16)] = (
                cnts[pl.ds(c * 16, 16)] + jnp.where(v > zeros16, ones16,
                                                    zeros16))

    for c in range(BPW // 16):
        qidx[pl.ds(c * 16, 16)] = xtbuf[pl.ds(HIST * BPW + c * 16, 16)]

    def remap16(ref, o):
        v = ref[pl.ds(o, 16)]
        ref[pl.ds(o, 16)] = jnp.where(v > zeros16, v, zrow16)

    def remap_x(s, carry):
        remap16(xbuf, s * 16)
        return carry

    lax.fori_loop(0, IPW // 16, remap_x, 0)

    pltpu.async_copy(tab_hbm.at[qidx], ebufs[0], sem).wait()
    pltpu.sync_copy(ebufs[0], q_out.at[pl.ds(base, BPW)])

    def start_g(idx_ref, t, k):
        pltpu.async_copy(tab_hbm.at[idx_ref.at[pl.ds(t * BPW, BPW)]],
                         ebufs[k], gsems[k])

    def drain_g(k):
        pltpu.make_async_copy(tab_hbm.at[pl.ds(0, BPW), :], ebufs[k],
                              gsems[k]).wait()

    NIT = (CH + NRING - 1) // NRING

    def emit(idx_ref, out_ref, wbase, hidden):
        for k in range(NRING):
            start_g(idx_ref, k, k)

        def step(i, carry):
            for k in range(NRING):
                t = NRING * i + k

                @pl.when(t < CH)
                def _(t=t, k=k):
                    drain_g(k)
                    pltpu.async_copy(
                        ebufs[k], out_ref.at[pl.ds(wbase + t * BPW, BPW), :],
                        wsems[k])

            hidden(i)

            for k in range(NRING):
                t = NRING * i + k

                @pl.when(t < CH)
                def _(t=t, k=k):
                    pltpu.make_async_copy(
                        ebufs[k], out_ref.at[pl.ds(wbase + t * BPW, BPW), :],
                        wsems[k]).wait()

                    @pl.when(t + NRING < CH)
                    def _():
                        start_g(idx_ref, t + NRING, k)

            return carry

        lax.fori_loop(0, NIT, step, 0)

    RSTEP = (IPW // 16 + NIT - 1) // NIT
    CSTEP = (CH + NIT - 1) // NIT

    def user_hidden(i):
        for s in range(RSTEP):
            o = (RSTEP * i + s) * 16

            @pl.when(o < IPW)
            def _(o=o):
                remap16(nbuf, o)

        for s in range(CSTEP):
            j = CSTEP * i + s

            @pl.when(j < HIST)
            def _(j=j):
                cstep(j)

    emit(xbuf, ub_out, base * HIST, user_hidden)
    pltpu.sync_copy(cnts, bl_out.at[pl.ds(base, BPW)])
    emit(nbuf, nub_out, base * HIST, lambda i: None)


@jax.jit
def _impl(x, neg_x, table):
    tab2 = jnp.concatenate(
        [table, jnp.zeros((1, EMBED), jnp.float32)], axis=0)
    xb = x[:, :HIST].reshape(-1)
    xpad = jnp.pad(x, ((0, 0), (0, XSTRIDE - (HIST + 1))))
    xtr = (xpad.T.reshape(XSTRIDE, NUM_WORKERS, BPW)
           .transpose(1, 0, 2).reshape(-1))
    negf = neg_x.reshape(-1)
    fn = pl.kernel(
        _sc_body,
        out_type=(
            jax.ShapeDtypeStruct((BATCH, EMBED), jnp.float32),
            jax.ShapeDtypeStruct((BATCH * HIST, EMBED), jnp.float32),
            jax.ShapeDtypeStruct((BATCH,), jnp.int32),
            jax.ShapeDtypeStruct((BATCH * HIST, EMBED), jnp.float32),
        ),
        mesh=plsc.VectorSubcoreMesh(core_axis_name="c", subcore_axis_name="s"),
        scratch_types=[
            pltpu.VMEM((IPW,), jnp.int32),
            pltpu.VMEM((BPW * XSTRIDE,), jnp.int32),
            pltpu.VMEM((IPW,), jnp.int32),
            [pltpu.VMEM((BPW, EMBED), jnp.float32) for _ in range(NRING)],
            pltpu.VMEM((BPW,), jnp.int32),
            pltpu.VMEM((BPW,), jnp.int32),
            pltpu.SemaphoreType.DMA,
            [pltpu.SemaphoreType.DMA for _ in range(NRING)],
            [pltpu.SemaphoreType.DMA for _ in range(NRING)],
        ],
    )
    q, ub, bl, nub = fn(xb, xtr, negf, tab2)
    return (q.reshape(BATCH, 1, EMBED), ub.reshape(BATCH, HIST, EMBED), bl,
            nub.reshape(BATCH, HIST, EMBED))


def kernel(x, neg_x, table):
    return _impl(x, neg_x, table)


# speedup vs baseline: 1.0339x; 1.0136x over previous
import jax
import jax.numpy as jnp
from jax import lax
from jax.experimental import pallas as pl
from jax.experimental.pallas import tpu as pltpu
from jax.experimental.pallas import tpu_sc as plsc

BATCH = 4096
HIST = 200
FEATURE_DIM = 100000
EMBED = 128
NUM_WORKERS = 32
BPW = BATCH // NUM_WORKERS
ZROW = FEATURE_DIM + 1
IPW = BPW * HIST
CH = IPW // BPW
NRING = 4
NIT = CH // NRING
RSTEP = (IPW // 16) // NIT


def _sc_body(xb_hbm, xq_hbm, neg_hbm, tab_hbm, q_out, ub_out, nub_out,
             xbuf, nbuf, ebufs, qidx, sem, gsems, wsems):
    wid = lax.axis_index("s") * 2 + lax.axis_index("c")
    base = wid * BPW

    pltpu.sync_copy(xb_hbm.at[pl.ds(base * HIST, IPW)], xbuf)
    pltpu.sync_copy(neg_hbm.at[pl.ds(base * HIST, IPW)], nbuf)
    pltpu.sync_copy(xq_hbm.at[pl.ds(base, BPW)], qidx)

    zeros16 = jnp.zeros((16,), jnp.int32)
    zrow16 = jnp.full((16,), ZROW, jnp.int32)

    def remap16(ref, o):
        v = ref[pl.ds(o, 16)]
        ref[pl.ds(o, 16)] = jnp.where(v > zeros16, v, zrow16)

    def remap_x(s, carry):
        remap16(xbuf, s * 16)
        return carry

    lax.fori_loop(0, IPW // 16, remap_x, 0)

    pltpu.async_copy(tab_hbm.at[qidx], ebufs[0], sem).wait()
    pltpu.sync_copy(ebufs[0], q_out.at[pl.ds(base, BPW)])

    def start_g(idx_ref, t, k):
        pltpu.async_copy(tab_hbm.at[idx_ref.at[pl.ds(t * BPW, BPW)]],
                         ebufs[k], gsems[k])

    def drain_g(k):
        pltpu.make_async_copy(tab_hbm.at[pl.ds(0, BPW), :], ebufs[k],
                              gsems[k]).wait()

    def emit(idx_ref, out_ref, wbase, hidden):
        for k in range(NRING):
            start_g(idx_ref, k, k)

        def step(i, carry):
            for k in range(NRING):
                t = NRING * i + k
                drain_g(k)
                pltpu.async_copy(
                    ebufs[k], out_ref.at[pl.ds(wbase + t * BPW, BPW), :],
                    wsems[k])

            hidden(i)

            for k in range(NRING):
                t = NRING * i + k
                pltpu.make_async_copy(
                    ebufs[k], out_ref.at[pl.ds(wbase + t * BPW, BPW), :],
                    wsems[k]).wait()

                @pl.when(t + NRING < CH)
                def _(t=t, k=k):
                    start_g(idx_ref, t + NRING, k)

            return carry

        lax.fori_loop(0, NIT, step, 0)

    def user_hidden(i):
        for s in range(RSTEP):
            remap16(nbuf, (RSTEP * i + s) * 16)

    emit(xbuf, ub_out, base * HIST, user_hidden)
    emit(nbuf, nub_out, base * HIST, lambda i: None)


def _tc_len_body(x_ref, o_ref):
    m = jnp.where(x_ref[...] > 0, jnp.int32(1), jnp.int32(0))
    o_ref[...] = jnp.sum(m, axis=1)


@jax.jit
def _impl(x, neg_x, table):
    tab2 = jnp.concatenate(
        [table, jnp.zeros((1, EMBED), jnp.float32)], axis=0)
    xb2d = x[:, :HIST]
    xb = xb2d.reshape(-1)
    xq = x[:, HIST]
    negf = neg_x.reshape(-1)

    fn = pl.kernel(
        _sc_body,
        out_type=(
            jax.ShapeDtypeStruct((BATCH, EMBED), jnp.float32),
            jax.ShapeDtypeStruct((BATCH * HIST, EMBED), jnp.float32),
            jax.ShapeDtypeStruct((BATCH * HIST, EMBED), jnp.float32),
        ),
        mesh=plsc.VectorSubcoreMesh(core_axis_name="c", subcore_axis_name="s"),
        scratch_types=[
            pltpu.VMEM((IPW,), jnp.int32),
            pltpu.VMEM((IPW,), jnp.int32),
            [pltpu.VMEM((BPW, EMBED), jnp.float32) for _ in range(NRING)],
            pltpu.VMEM((BPW,), jnp.int32),
            pltpu.SemaphoreType.DMA,
            [pltpu.SemaphoreType.DMA for _ in range(NRING)],
            [pltpu.SemaphoreType.DMA for _ in range(NRING)],
        ],
    )
    q, ub, nub = fn(xb, xq, negf, tab2)

    bl = pl.pallas_call(
        _tc_len_body,
        grid=(8,),
        in_specs=[pl.BlockSpec((BATCH // 8, HIST), lambda i: (i, 0))],
        out_specs=pl.BlockSpec((BATCH // 8,), lambda i: (i,)),
        out_shape=jax.ShapeDtypeStruct((BATCH,), jnp.int32),
    )(xb2d)

    return (q.reshape(BATCH, 1, EMBED), ub.reshape(BATCH, HIST, EMBED), bl,
            nub.reshape(BATCH, HIST, EMBED))


def kernel(x, neg_x, table):
    return _impl(x, neg_x, table)
